# Initial kernel scaffold; baseline (speedup 1.0000x reference)
import jax
import jax.numpy as jnp
from jax.experimental import pallas as pl


def kernel(user_emb, user_offset_emb, item_emb, item_offset_emb, graph, center_W1, center_b1, center_W2, center_b2, offset_W1, offset_b1, offset_W2, offset_b2):
    raise NotImplementedError("write your pallas kernel here")


# speedup vs baseline: 1.4369x; 1.4369x over previous
import functools

import jax
import jax.numpy as jnp
from jax.experimental import pallas as pl
from jax.experimental.pallas import tpu as pltpu

N_USERS = 3000
N_ITEMS = 5000
N_ENT = 7000
N_NODES = N_USERS + N_ENT
D = 256
N_HOPS = 2
E = 160000

BE = 2000
DUMP = 13000


def _edge_mlp_body(h_ref, ho_ref, w1c_ref, b1c_ref, w2c_ref, b2c_ref,
                   w1o_ref, b1o_ref, h2_ref, l1_ref):
    x = h_ref[...]
    h = jnp.maximum(
        jnp.dot(x, w1c_ref[...], preferred_element_type=jnp.float32)
        + b1c_ref[...], 0.0)
    h2_ref[...] = (jnp.dot(h, w2c_ref[...], preferred_element_type=jnp.float32)
                   + b2c_ref[...])
    ho = jnp.maximum(ho_ref[...], 0.0)
    l1_ref[...] = jnp.maximum(
        jnp.dot(ho, w1o_ref[...], preferred_element_type=jnp.float32)
        + b1o_ref[...], 0.0)


def _edge_mlp(hist, ho, w1c_t, b1c, w2c_t, b2c, w1o_t, b1o):
    n = hist.shape[0]
    grid = n // BE
    row_spec = pl.BlockSpec((BE, D), lambda i: (i, 0))
    full = pl.BlockSpec((D, D), lambda i: (0, 0))
    bias = pl.BlockSpec((1, D), lambda i: (0, 0))
    return pl.pallas_call(
        _edge_mlp_body,
        grid=(grid,),
        in_specs=[row_spec, row_spec, full, bias, full, bias, full, bias],
        out_specs=[row_spec, row_spec],
        out_shape=[jax.ShapeDtypeStruct((n, D), jnp.float32),
                   jax.ShapeDtypeStruct((n, D), jnp.float32)],
    )(hist, ho, w1c_t, b1c.reshape(1, D), w2c_t, b2c.reshape(1, D),
      w1o_t, b1o.reshape(1, D))


def _dense_body(act, x_ref, w_ref, b_ref, o_ref):
    y = (jnp.dot(x_ref[...], w_ref[...], preferred_element_type=jnp.float32)
         + b_ref[...])
    if act == "relu":
        y = jnp.maximum(y, 0.0)
    elif act == "sigmoid":
        y = jax.nn.sigmoid(y)
    o_ref[...] = y


def _dense(x, w_t, b, act, blk):
    n = x.shape[0]
    grid = n // blk
    row_spec = pl.BlockSpec((blk, D), lambda i: (i, 0))
    return pl.pallas_call(
        functools.partial(_dense_body, act),
        grid=(grid,),
        in_specs=[row_spec, pl.BlockSpec((D, D), lambda i: (0, 0)),
                  pl.BlockSpec((1, D), lambda i: (0, 0))],
        out_specs=row_spec,
        out_shape=jax.ShapeDtypeStruct((n, D), jnp.float32),
    )(x, w_t, b.reshape(1, D))


def kernel(user_emb, user_offset_emb, item_emb, item_offset_emb, graph,
           center_W1, center_b1, center_W2, center_b2,
           offset_W1, offset_b1, offset_W2, offset_b2):
    head = graph[0]
    tail = graph[1]

    m1 = (tail >= N_USERS) & (head < N_USERS) & (tail < N_USERS + N_ITEMS)
    m2 = (head < N_USERS) & (tail >= N_USERS + N_ITEMS)
    rest = (head >= N_USERS) | m2
    m4 = head >= N_USERS + N_ITEMS

    i1 = jnp.where(m1, head, DUMP)
    i2 = jnp.where(rest, head + N_USERS, DUMP)
    i_min = jnp.where(m1, head, jnp.where(m4, head + N_USERS, DUMP))
    i_max = jnp.where(rest & ~m4, head + N_USERS, DUMP)

    w1c_t = center_W1.T
    w2c_t = center_W2.T
    w1o_t = offset_W1.T
    w2o_t = offset_W2.T

    all_embs = jnp.concatenate([user_emb, item_emb], axis=0)
    all_off = jax.nn.relu(
        jnp.concatenate([user_offset_emb, item_offset_emb], axis=0))

    for _ in range(N_HOPS):
        hist = all_embs[tail]
        hist_off = jnp.maximum(all_off[tail], 0.0)

        h2, l1 = _edge_mlp(hist, hist_off, w1c_t, center_b1, w2c_t, center_b2,
                           w1o_t, offset_b1)

        m = jax.ops.segment_max(h2, head, num_segments=N_NODES)
        m = jnp.where(jnp.isfinite(m), m, 0.0)
        e = jnp.exp(h2 - m[head])
        se = jax.ops.segment_sum(e, head, num_segments=N_NODES)
        sew = jax.ops.segment_sum(e * hist, head, num_segments=N_NODES)
        agg = sew / (se + 1e-16)

        s1 = (jax.ops.segment_sum(l1, i1, num_segments=DUMP + 1)
              + jax.ops.segment_sum(l1, i2, num_segments=DUMP + 1))
        ones = jnp.ones((E,), jnp.float32)
        cnt = (jax.ops.segment_sum(ones, i1, num_segments=DUMP + 1)
               + jax.ops.segment_sum(ones, i2, num_segments=DUMP + 1))
        a_min = jax.ops.segment_min(hist_off, i_min, num_segments=DUMP + 1)
        a_max = jax.ops.segment_max(hist_off, i_max, num_segments=DUMP + 1)

        s1 = s1[:DUMP]
        cnt = cnt[:DUMP]
        off = jnp.concatenate([a_min[:N_USERS],
                               a_max[N_USERS:11000],
                               a_min[11000:DUMP]], axis=0)

        mean = s1 / jnp.maximum(cnt, 1.0)[:, None]
        gate = _dense(mean, w2o_t, offset_b2, "sigmoid", 1000)
        off = jnp.where((cnt > 0)[:, None], off, 0.0)
        off = off * gate

        iu = off[:N_USERS]
        ut = off[N_USERS:2 * N_USERS]
        it = off[2 * N_USERS:2 * N_USERS + N_ITEMS]
        tg = off[11000:DUMP]

        uo2 = jnp.concatenate([iu, ut], axis=0)
        l1u = _dense(uo2, w1o_t, offset_b1, "relu", 1000)
        mean2 = 0.5 * (l1u[:N_USERS] + l1u[N_USERS:])
        gate2 = _dense(mean2, w2o_t, offset_b2, "sigmoid", 1000)
        uo = jax.nn.relu(jnp.maximum(iu, ut) * gate2)

        agg_off = jnp.concatenate([uo, it, tg], axis=0)

        nrm = jnp.sqrt(jnp.sum(agg * agg, axis=1, keepdims=True))
        all_embs = agg / jnp.maximum(nrm, 1e-12)
        all_off = agg_off

    return all_embs, all_off


# speedup vs baseline: 1.8480x; 1.2861x over previous
import functools

import jax
import jax.numpy as jnp
from jax.experimental import pallas as pl
from jax.experimental.pallas import tpu as pltpu

N_USERS = 3000
N_ITEMS = 5000
N_ENT = 7000
N_NODES = N_USERS + N_ENT
D = 256
N_HOPS = 2
E = 160000

BE = 2000
DUMP = 13000


def _edge_mlp_body(h_ref, ho_ref, w1c_ref, b1c_ref, w2c_ref, b2c_ref,
                   w1o_ref, b1o_ref, h2_ref, l1_ref):
    x = h_ref[...]
    h = jnp.maximum(
        jnp.dot(x, w1c_ref[...], preferred_element_type=jnp.float32)
        + b1c_ref[...], 0.0)
    h2_ref[...] = (jnp.dot(h, w2c_ref[...], preferred_element_type=jnp.float32)
                   + b2c_ref[...])
    ho = jnp.maximum(ho_ref[...], 0.0)
    l1_ref[...] = jnp.maximum(
        jnp.dot(ho, w1o_ref[...], preferred_element_type=jnp.float32)
        + b1o_ref[...], 0.0)


def _edge_mlp(hist, ho, w1c_t, b1c, w2c_t, b2c, w1o_t, b1o):
    n = hist.shape[0]
    grid = n // BE
    row_spec = pl.BlockSpec((BE, D), lambda i: (i, 0))
    full = pl.BlockSpec((D, D), lambda i: (0, 0))
    bias = pl.BlockSpec((1, D), lambda i: (0, 0))
    return pl.pallas_call(
        _edge_mlp_body,
        grid=(grid,),
        in_specs=[row_spec, row_spec, full, bias, full, bias, full, bias],
        out_specs=[row_spec, row_spec],
        out_shape=[jax.ShapeDtypeStruct((n, D), jnp.float32),
                   jax.ShapeDtypeStruct((n, D), jnp.float32)],
    )(hist, ho, w1c_t, b1c.reshape(1, D), w2c_t, b2c.reshape(1, D),
      w1o_t, b1o.reshape(1, D))


def _dense_body(act, x_ref, w_ref, b_ref, o_ref):
    y = (jnp.dot(x_ref[...], w_ref[...], preferred_element_type=jnp.float32)
         + b_ref[...])
    if act == "relu":
        y = jnp.maximum(y, 0.0)
    elif act == "sigmoid":
        y = jax.nn.sigmoid(y)
    o_ref[...] = y


def _dense(x, w_t, b, act, blk):
    n = x.shape[0]
    grid = n // blk
    row_spec = pl.BlockSpec((blk, D), lambda i: (i, 0))
    return pl.pallas_call(
        functools.partial(_dense_body, act),
        grid=(grid,),
        in_specs=[row_spec, pl.BlockSpec((D, D), lambda i: (0, 0)),
                  pl.BlockSpec((1, D), lambda i: (0, 0))],
        out_specs=row_spec,
        out_shape=jax.ShapeDtypeStruct((n, D), jnp.float32),
    )(x, w_t, b.reshape(1, D))


def kernel(user_emb, user_offset_emb, item_emb, item_offset_emb, graph,
           center_W1, center_b1, center_W2, center_b2,
           offset_W1, offset_b1, offset_W2, offset_b2):
    head = graph[0]
    tail = graph[1]

    m1 = (tail >= N_USERS) & (head < N_USERS) & (tail < N_USERS + N_ITEMS)
    m2 = (head < N_USERS) & (tail >= N_USERS + N_ITEMS)
    rest = (head >= N_USERS) | m2
    m4 = head >= N_USERS + N_ITEMS

    i_sum = jnp.where(m1, head, jnp.where(rest, head + N_USERS, DUMP))
    is_min = m1 | m4

    w1c_t = center_W1.T
    w2c_t = center_W2.T
    w1o_t = offset_W1.T
    w2o_t = offset_W2.T

    all_embs = jnp.concatenate([user_emb, item_emb], axis=0)
    all_off = jax.nn.relu(
        jnp.concatenate([user_offset_emb, item_offset_emb], axis=0))

    cnt = jax.ops.segment_sum(jnp.ones((E,), jnp.float32), i_sum,
                              num_segments=DUMP + 1)[:DUMP]

    for _ in range(N_HOPS):
        gathered = jnp.concatenate([all_embs, all_off], axis=1)[tail]
        hist = gathered[:, :D]
        hist_off = jnp.maximum(gathered[:, D:], 0.0)

        h2, l1 = _edge_mlp(hist, hist_off, w1c_t, center_b1, w2c_t, center_b2,
                           w1o_t, offset_b1)

        m = jax.ops.segment_max(h2, head, num_segments=N_NODES)
        m = jnp.where(jnp.isfinite(m), m, 0.0)
        e = jnp.exp(h2 - m[head])
        es = jax.ops.segment_sum(jnp.concatenate([e, e * hist], axis=1),
                                 head, num_segments=N_NODES)
        agg = es[:, D:] / (es[:, :D] + 1e-16)

        s1 = jax.ops.segment_sum(l1, i_sum, num_segments=DUMP + 1)[:DUMP]
        z = jnp.where(is_min[:, None], -hist_off, hist_off)
        a = jax.ops.segment_max(z, i_sum, num_segments=DUMP + 1)
        off = jnp.concatenate([-a[:N_USERS],
                               a[N_USERS:11000],
                               -a[11000:DUMP]], axis=0)

        mean = s1 / jnp.maximum(cnt, 1.0)[:, None]
        gate = _dense(mean, w2o_t, offset_b2, "sigmoid", 1000)
        off = jnp.where((cnt > 0)[:, None], off, 0.0)
        off = off * gate

        iu = off[:N_USERS]
        ut = off[N_USERS:2 * N_USERS]
        it = off[2 * N_USERS:2 * N_USERS + N_ITEMS]
        tg = off[11000:DUMP]

        uo2 = jnp.concatenate([iu, ut], axis=0)
        l1u = _dense(uo2, w1o_t, offset_b1, "relu", 1000)
        mean2 = 0.5 * (l1u[:N_USERS] + l1u[N_USERS:])
        gate2 = _dense(mean2, w2o_t, offset_b2, "sigmoid", 1000)
        uo = jax.nn.relu(jnp.maximum(iu, ut) * gate2)

        agg_off = jnp.concatenate([uo, it, tg], axis=0)

        nrm = jnp.sqrt(jnp.sum(agg * agg, axis=1, keepdims=True))
        all_embs = agg / jnp.maximum(nrm, 1e-12)
        all_off = agg_off

    return all_embs, all_off


# speedup vs baseline: 1.9290x; 1.0439x over previous
import functools

import jax
import jax.numpy as jnp
from jax import lax
from jax.experimental import pallas as pl
from jax.experimental.pallas import tpu as pltpu
from jax.experimental.pallas import tpu_sc as plsc

N_USERS = 3000
N_ITEMS = 5000
N_ENT = 7000
N_NODES = N_USERS + N_ENT
D = 256
N_HOPS = 2
E = 160000

BE = 2000
DUMP = 13000


def _edge_mlp_body(g_ref, w1c_ref, b1c_ref, w2c_ref, b2c_ref,
                   w1o_ref, b1o_ref, h2_ref, l1_ref):
    x = g_ref[:, :D]
    h = jnp.maximum(
        jnp.dot(x, w1c_ref[...], preferred_element_type=jnp.float32)
        + b1c_ref[...], 0.0)
    h2_ref[...] = (jnp.dot(h, w2c_ref[...], preferred_element_type=jnp.float32)
                   + b2c_ref[...])
    ho = jnp.maximum(g_ref[:, D:], 0.0)
    l1_ref[...] = jnp.maximum(
        jnp.dot(ho, w1o_ref[...], preferred_element_type=jnp.float32)
        + b1o_ref[...], 0.0)


def _edge_mlp(gathered, w1c_t, b1c, w2c_t, b2c, w1o_t, b1o):
    n = gathered.shape[0]
    grid = n // BE
    row_spec = pl.BlockSpec((BE, D), lambda i: (i, 0))
    full = pl.BlockSpec((D, D), lambda i: (0, 0))
    bias = pl.BlockSpec((1, D), lambda i: (0, 0))
    return pl.pallas_call(
        _edge_mlp_body,
        grid=(grid,),
        in_specs=[pl.BlockSpec((BE, 2 * D), lambda i: (i, 0)),
                  full, bias, full, bias, full, bias],
        out_specs=[row_spec, row_spec],
        out_shape=[jax.ShapeDtypeStruct((n, D), jnp.float32),
                   jax.ShapeDtypeStruct((n, D), jnp.float32)],
    )(gathered, w1c_t, b1c.reshape(1, D), w2c_t, b2c.reshape(1, D),
      w1o_t, b1o.reshape(1, D))


_NW = 32
_EPW = E // _NW


@functools.partial(jax.jit, static_argnames=("width", "grp"))
def _sc_gather(table, idx, width, grp):
    mesh = plsc.VectorSubcoreMesh(core_axis_name="c", subcore_axis_name="s")

    @functools.partial(
        pl.kernel, mesh=mesh,
        out_type=jax.ShapeDtypeStruct((E, width), jnp.float32),
        scratch_types=[
            pltpu.VMEM((_EPW,), jnp.int32),
            pltpu.VMEM((grp, width), jnp.float32),
            pltpu.SemaphoreType.DMA,
        ],
    )
    def k(table_hbm, idx_hbm, out_hbm, idx_v, buf, sem):
        wid = lax.axis_index("s") * 2 + lax.axis_index("c")
        base = wid * _EPW
        pltpu.sync_copy(idx_hbm.at[pl.ds(base, _EPW)], idx_v)

        def body(c, carry):
            pltpu.async_copy(
                table_hbm.at[idx_v.at[pl.ds(c * grp, grp)]], buf, sem
            ).wait()
            pltpu.sync_copy(buf, out_hbm.at[pl.ds(base + c * grp, grp)])
            return carry

        lax.fori_loop(0, _EPW // grp, body, 0)

    return k(table, idx)


def _dense_body(act, x_ref, w_ref, b_ref, o_ref):
    y = (jnp.dot(x_ref[...], w_ref[...], preferred_element_type=jnp.float32)
         + b_ref[...])
    if act == "relu":
        y = jnp.maximum(y, 0.0)
    elif act == "sigmoid":
        y = jax.nn.sigmoid(y)
    o_ref[...] = y


def _dense(x, w_t, b, act, blk):
    n = x.shape[0]
    grid = n // blk
    row_spec = pl.BlockSpec((blk, D), lambda i: (i, 0))
    return pl.pallas_call(
        functools.partial(_dense_body, act),
        grid=(grid,),
        in_specs=[row_spec, pl.BlockSpec((D, D), lambda i: (0, 0)),
                  pl.BlockSpec((1, D), lambda i: (0, 0))],
        out_specs=row_spec,
        out_shape=jax.ShapeDtypeStruct((n, D), jnp.float32),
    )(x, w_t, b.reshape(1, D))


def kernel(user_emb, user_offset_emb, item_emb, item_offset_emb, graph,
           center_W1, center_b1, center_W2, center_b2,
           offset_W1, offset_b1, offset_W2, offset_b2):
    head = graph[0]
    tail = graph[1]

    m1 = (tail >= N_USERS) & (head < N_USERS) & (tail < N_USERS + N_ITEMS)
    m2 = (head < N_USERS) & (tail >= N_USERS + N_ITEMS)
    rest = (head >= N_USERS) | m2
    m4 = head >= N_USERS + N_ITEMS

    i_sum = jnp.where(m1, head, jnp.where(rest, head + N_USERS, DUMP))
    is_min = m1 | m4

    w1c_t = center_W1.T
    w2c_t = center_W2.T
    w1o_t = offset_W1.T
    w2o_t = offset_W2.T

    all_embs = jnp.concatenate([user_emb, item_emb], axis=0)
    all_off = jax.nn.relu(
        jnp.concatenate([user_offset_emb, item_offset_emb], axis=0))

    cnt = jax.ops.segment_sum(jnp.ones((E,), jnp.float32), i_sum,
                              num_segments=DUMP + 1)[:DUMP]

    for _ in range(N_HOPS):
        table = jnp.concatenate([all_embs, all_off], axis=1)
        gathered = _sc_gather(table, tail, 512, 200)
        hist = gathered[:, :D]
        hist_off = jnp.maximum(gathered[:, D:], 0.0)

        h2, l1 = _edge_mlp(gathered, w1c_t, center_b1, w2c_t, center_b2,
                           w1o_t, offset_b1)

        m = jax.ops.segment_max(h2, head, num_segments=N_NODES)
        m = jnp.where(jnp.isfinite(m), m, 0.0)
        e = jnp.exp(h2 - _sc_gather(m, head, D, 200))
        es = jax.ops.segment_sum(jnp.concatenate([e, e * hist], axis=1),
                                 head, num_segments=N_NODES)
        agg = es[:, D:] / (es[:, :D] + 1e-16)

        s1 = jax.ops.segment_sum(l1, i_sum, num_segments=DUMP + 1)[:DUMP]
        z = jnp.where(is_min[:, None], -hist_off, hist_off)
        a = jax.ops.segment_max(z, i_sum, num_segments=DUMP + 1)
        off = jnp.concatenate([-a[:N_USERS],
                               a[N_USERS:11000],
                               -a[11000:DUMP]], axis=0)

        mean = s1 / jnp.maximum(cnt, 1.0)[:, None]
        gate = _dense(mean, w2o_t, offset_b2, "sigmoid", 1000)
        off = jnp.where((cnt > 0)[:, None], off, 0.0)
        off = off * gate

        iu = off[:N_USERS]
        ut = off[N_USERS:2 * N_USERS]
        it = off[2 * N_USERS:2 * N_USERS + N_ITEMS]
        tg = off[11000:DUMP]

        uo2 = jnp.concatenate([iu, ut], axis=0)
        l1u = _dense(uo2, w1o_t, offset_b1, "relu", 1000)
        mean2 = 0.5 * (l1u[:N_USERS] + l1u[N_USERS:])
        gate2 = _dense(mean2, w2o_t, offset_b2, "sigmoid", 1000)
        uo = jax.nn.relu(jnp.maximum(iu, ut) * gate2)

        agg_off = jnp.concatenate([uo, it, tg], axis=0)

        nrm = jnp.sqrt(jnp.sum(agg * agg, axis=1, keepdims=True))
        all_embs = agg / jnp.maximum(nrm, 1e-12)
        all_off = agg_off

    return all_embs, all_off
